# Initial kernel scaffold; baseline (speedup 1.0000x reference)
import jax
import jax.numpy as jnp
from jax.experimental import pallas as pl


def kernel(gt_inds, anchor_gt_inds, gt_bboxes, proposal_list):
    raise NotImplementedError("write your pallas kernel here")


# speedup vs baseline: 1619.0457x; 1619.0457x over previous
import functools

import jax
import jax.numpy as jnp
from jax import lax
from jax.experimental import pallas as pl
from jax.experimental.pallas import tpu as pltpu

NMS_THR = 0.5
EPS = 1e-06
_N = 5000
_G = 100
_LANES = 128
_ROWS = 40
_NP = _ROWS * _LANES

_NEG_INF = float("-inf")


def _nms_kernel(x1_ref, y1_ref, x2_ref, y2_ref, s_ref, g_ref, gt_ref,
                push_ref, pull_ref):
    i32 = jnp.int32
    f32 = jnp.float32

    x1 = x1_ref[...]
    y1 = y1_ref[...]
    x2 = x2_ref[...]
    y2 = y2_ref[...]
    s = s_ref[...]
    gv = g_ref[...]

    row_iota = lax.broadcasted_iota(i32, (_ROWS, _LANES), 0)
    lane_iota = lax.broadcasted_iota(i32, (_ROWS, _LANES), 1)
    flat = row_iota * _LANES + lane_iota
    lane128 = lax.broadcasted_iota(i32, (1, _LANES), 1)

    area = (x2 - x1 + 1.0) * (y2 - y1 + 1.0)

    def gsel(k, carry):
        gx1, gy1, gx2, gy2 = carry
        m = gv == k
        return (jnp.where(m, gt_ref[k, 0], gx1),
                jnp.where(m, gt_ref[k, 1], gy1),
                jnp.where(m, gt_ref[k, 2], gx2),
                jnp.where(m, gt_ref[k, 3], gy2))

    z = jnp.zeros((_ROWS, _LANES), f32)
    gx1, gy1, gx2, gy2 = lax.fori_loop(0, _G, gsel, (z, z, z, z))
    garea = (gx2 - gx1 + 1.0) * (gy2 - gy1 + 1.0)

    alive0 = (gv >= 0).astype(i32)
    live0 = jnp.sum(alive0) > 0
    rec0 = jnp.full((1, _LANES), -1, i32)

    def cond(st):
        return st[0]

    def body(st):
        _, alive_i, rec, tot_pull, tot_push, pull_cnt, push_cnt = st
        alive = alive_i > 0

        ms = jnp.where(alive, s, _NEG_INF)
        m = jnp.max(ms)
        i = jnp.max(jnp.where(ms == m, flat, -1))

        r = i // _LANES
        c = i - r * _LANES
        coh = lane128 == c

        def ext_f(ref):
            return jnp.sum(jnp.where(coh, ref[pl.ds(r, 1), :], 0.0))

        x1i = ext_f(x1_ref)
        y1i = ext_f(y1_ref)
        x2i = ext_f(x2_ref)
        y2i = ext_f(y2_ref)
        si = ext_f(s_ref)
        gi = jnp.sum(jnp.where(coh, g_ref[pl.ds(r, 1), :], 0))

        area_i = (x2i - x1i + 1.0) * (y2i - y1i + 1.0)

        w = jnp.maximum(jnp.minimum(x2i, x2) - jnp.maximum(x1i, x1) + 1.0, 0.0)
        h = jnp.maximum(jnp.minimum(y2i, y2) - jnp.maximum(y1i, y1) + 1.0, 0.0)
        ovl = w * h
        row = ovl / (area_i + area - ovl)

        alive2 = alive & (flat != i)
        remaining = jnp.sum(alive2.astype(i32))

        goh = lane128 == gi
        rep = jnp.sum(jnp.where(goh, rec, 0))
        has = rep >= 0
        lrow = jnp.log(jnp.maximum(row, EPS))
        pv = jnp.sum(jnp.where(flat == rep, lrow, 0.0))
        pull = jnp.where(has, -pv * si, 0.0)
        rec_new = jnp.where(goh & (rep < 0), i, rec)

        ga_x1 = gt_ref[gi, 0]
        ga_y1 = gt_ref[gi, 1]
        ga_x2 = gt_ref[gi, 2]
        ga_y2 = gt_ref[gi, 3]
        ga_area = (ga_x2 - ga_x1 + 1.0) * (ga_y2 - ga_y1 + 1.0)
        gw = jnp.maximum(jnp.minimum(ga_x2, gx2) - jnp.maximum(ga_x1, gx1) + 1.0, 0.0)
        gh = jnp.maximum(jnp.minimum(ga_y2, gy2) - jnp.maximum(ga_y1, gy1) + 1.0, 0.0)
        govl = gw * gh
        giou = govl / (ga_area + garea - govl)

        ov = alive2 & (row > NMS_THR)
        pm2 = ov & (gv != gi) & (row > giou)
        cnt2 = jnp.sum(pm2.astype(i32))
        plv = -jnp.log(1.0 + NMS_THR - row) * s
        push_sum = jnp.sum(jnp.where(pm2, plv, 0.0))
        push = jnp.where(cnt2 > 0, push_sum / cnt2.astype(f32), 0.0)

        cont = remaining > 0
        tot_pull = tot_pull + jnp.where(cont, pull, 0.0)
        tot_push = tot_push + jnp.where(cont, push, 0.0)
        pull_cnt = pull_cnt + jnp.where(has, 1, 0)
        push_cnt = push_cnt + jnp.where(cont, cnt2, 0)

        alive_new = (alive2 & (row <= NMS_THR)).astype(i32)
        live_new = jnp.sum(alive_new) > 0
        return (live_new, alive_new, rec_new, tot_pull, tot_push,
                pull_cnt, push_cnt)

    init = (live0, alive0, rec0, jnp.float32(0.0), jnp.float32(0.0),
            jnp.int32(0), jnp.int32(0))
    st = lax.while_loop(cond, body, init)
    _, _, _, tot_pull, tot_push, pull_cnt, push_cnt = st

    push_ref[0, 0] = tot_push / (push_cnt.astype(f32) + EPS)
    pull_ref[0, 0] = tot_pull / (pull_cnt.astype(f32) + EPS)


@functools.partial(jax.jit, static_argnums=())
def _run(g0, gt, props):
    pad = _NP - _N
    p = jnp.pad(props, ((0, pad), (0, 0)))
    x1 = p[:, 0].reshape(_ROWS, _LANES)
    y1 = p[:, 1].reshape(_ROWS, _LANES)
    x2 = p[:, 2].reshape(_ROWS, _LANES)
    y2 = p[:, 3].reshape(_ROWS, _LANES)
    s = p[:, 4].reshape(_ROWS, _LANES)
    g = jnp.pad(g0.astype(jnp.int32), (0, pad), constant_values=-1)
    g = g.reshape(_ROWS, _LANES)

    vmem = pl.BlockSpec(memory_space=pltpu.VMEM)
    smem = pl.BlockSpec(memory_space=pltpu.SMEM)
    push, pull = pl.pallas_call(
        _nms_kernel,
        in_specs=[vmem, vmem, vmem, vmem, vmem, vmem, smem],
        out_specs=[smem, smem],
        out_shape=[jax.ShapeDtypeStruct((1, 1), jnp.float32),
                   jax.ShapeDtypeStruct((1, 1), jnp.float32)],
    )(x1, y1, x2, y2, s, g, gt)
    return push.reshape(()), pull.reshape(())


def kernel(gt_inds, anchor_gt_inds, gt_bboxes, proposal_list):
    g0 = anchor_gt_inds[0]
    gt = gt_bboxes[0].astype(jnp.float32)
    props = proposal_list[0].astype(jnp.float32)
    push, pull = _run(g0, gt, props)
    return (push, pull)
